# baseline (speedup 1.0000x reference)
import functools

import jax
import jax.numpy as jnp
from jax import lax
from jax.experimental import pallas as pl
from jax.experimental.pallas import tpu as pltpu
from jax.experimental.pallas import tpu_sc as plsc

_NUM_CORES = 2
_NUM_SUBCORES = 16
_NUM_WORKERS = _NUM_CORES * _NUM_SUBCORES
_LANES = 16
_BC = 128
_TD = 4


def _gather_call(S, V, interpret=False):
    mesh = plsc.VectorSubcoreMesh(core_axis_name="c", subcore_axis_name="s")

    row_words = _TD * 8 * _BC

    @functools.partial(
        pl.kernel,
        mesh=mesh,
        out_type=jax.ShapeDtypeStruct((S, _NUM_WORKERS * row_words),
                                      jnp.float32),
        scratch_types=(
            [pltpu.VMEM((S, _BC), jnp.int32)]
            + [pltpu.VMEM((_BC, 32), jnp.float32) for _ in range(2)]
            + [pltpu.VMEM((row_words,), jnp.float32) for _ in range(2)]
            + [pltpu.SemaphoreType.DMA for _ in range(4)]
        ),
        compiler_params=pltpu.CompilerParams(use_tc_tiling_on_sc=False,
                                             needs_layout_passes=False),
        interpret=interpret,
    )
    def gather_kernel(table_hbm, ids_hbm, out_hbm, idx_all, r0, r1, o0, o1,
                      sg0, sg1, so0, so1):
        rows_v = (r0, r1)
        out_v = (o0, o1)
        s_g = (sg0, sg1)
        s_o = (so0, so1)
        w = lax.axis_index("s") * _NUM_CORES + lax.axis_index("c")

        pltpu.sync_copy(ids_hbm.at[:, w, :], idx_all)

        def start_gather(b, s):
            pltpu.async_copy(table_hbm.at[idx_all.at[s]], rows_v[b], s_g[b])

        def wait_gather(b):
            pltpu.make_async_copy(table_hbm.at[idx_all.at[0]],
                                  rows_v[b], s_g[b]).wait()

        def start_out(b, s):
            for td in range(_TD):
                pltpu.async_copy(
                    out_v[b].at[pl.ds(td * 1024, 1024)],
                    out_hbm.at[s, pl.ds(td * _NUM_WORKERS * 1024 + w * 1024,
                                        1024)],
                    s_o[b])

        def wait_out(b):
            for td in range(_TD):
                pltpu.make_async_copy(out_v[b].at[pl.ds(td * 1024, 1024)],
                                      out_hbm.at[0, pl.ds(td * 1024, 1024)],
                                      s_o[b]).wait()

        lane = lax.broadcasted_iota(jnp.int32, (_LANES,), 0)
        base = ((lane // 8) * 1024 + (lane % 8) * _BC)

        def transpose(b):
            for h in range(2):
                base_h = base + (h * 2) * 1024
                for bc in range(_BC):
                    vals = rows_v[b][bc, pl.ds(h * _LANES, _LANES)]
                    plsc.store_scatter(out_v[b], [base_h + bc], vals)

        start_gather(0, 0)

        def pair(p, _):
            for b in range(2):
                s = 2 * p + b
                wait_gather(b)

                @pl.when(s + 1 < S)
                def _():
                    start_gather(1 - b, s + 1)

                @pl.when(s >= 2)
                def _():
                    wait_out(b)

                transpose(b)
                start_out(b, s)
            return ()

        lax.fori_loop(0, S // 2, pair, ())
        wait_out(0)
        wait_out(1)

    return gather_kernel


def kernel(input_ids, table):
    batch, seq = input_ids.shape
    V, D = table.shape
    ids3 = input_ids.T.reshape(seq, batch // _BC, _BC).astype(jnp.int32)
    out2 = _gather_call(seq, V)(table, ids3)
    out5 = out2.reshape(seq, _TD, batch // _BC, 8, _BC)
    return out5.transpose(2, 4, 0, 1, 3).reshape(batch, seq, D)


# speedup vs baseline: 1.0644x; 1.0644x over previous
import functools

import jax
import jax.numpy as jnp
from jax import lax
from jax.experimental import pallas as pl
from jax.experimental.pallas import tpu as pltpu
from jax.experimental.pallas import tpu_sc as plsc

_NUM_CORES = 2
_NUM_SUBCORES = 16
_NUM_WORKERS = _NUM_CORES * _NUM_SUBCORES
_LANES = 16
_BC = 128
_TD = 4
_PRE = 4


def _gather_call(S, V):
    row_words = _TD * 8 * _BC
    mesh = plsc.VectorSubcoreMesh(core_axis_name="c", subcore_axis_name="s")

    @functools.partial(
        pl.kernel,
        mesh=mesh,
        out_type=jax.ShapeDtypeStruct((S, _NUM_WORKERS * row_words),
                                      jnp.float32),
        scratch_types=(
            [pltpu.VMEM((S, _BC), jnp.int32)]
            + [pltpu.VMEM((_BC, 32), jnp.float32) for _ in range(2)]
            + [pltpu.VMEM((row_words,), jnp.float32) for _ in range(2)]
            + [pltpu.SemaphoreType.DMA for _ in range(4)]
        ),
        compiler_params=pltpu.CompilerParams(use_tc_tiling_on_sc=False,
                                             needs_layout_passes=False),
    )
    def gather_kernel(table_hbm, ids_hbm, out_hbm, idx_all, r0, r1, o0, o1,
                      sg0, sg1, so0, so1):
        rows_v = (r0, r1)
        out_v = (o0, o1)
        s_g = (sg0, sg1)
        s_o = (so0, so1)
        w = lax.axis_index("s") * _NUM_CORES + lax.axis_index("c")

        pltpu.sync_copy(ids_hbm.at[:, w, :], idx_all)

        def start_gather(b, s):
            pltpu.async_copy(table_hbm.at[idx_all.at[s]], rows_v[b], s_g[b])

        def wait_gather(b):
            pltpu.make_async_copy(table_hbm.at[idx_all.at[0]],
                                  rows_v[b], s_g[b]).wait()

        def start_out(b, s):
            for td in range(_TD):
                pltpu.async_copy(
                    out_v[b].at[pl.ds(td * 1024, 1024)],
                    out_hbm.at[s, pl.ds(td * _NUM_WORKERS * 1024 + w * 1024,
                                        1024)],
                    s_o[b])

        def wait_out(b):
            for td in range(_TD):
                pltpu.make_async_copy(out_v[b].at[pl.ds(td * 1024, 1024)],
                                      out_hbm.at[0, pl.ds(td * 1024, 1024)],
                                      s_o[b]).wait()

        lane = lax.broadcasted_iota(jnp.int32, (_LANES,), 0)
        bases = [(lane // 8 + 2 * h) * 1024 + (lane % 8) * _BC
                 for h in range(2)]

        def transpose(b):
            def load(bc):
                return [rows_v[b][bc, pl.ds(h * _LANES, _LANES)]
                        for h in range(2)]

            def store(bc, vals):
                for h in range(2):
                    plsc.store_scatter(out_v[b], [bases[h] + bc], vals[h])

            pipe = [load(bc) for bc in range(_PRE)]
            for bc in range(_BC):
                if bc + _PRE < _BC:
                    pipe.append(load(bc + _PRE))
                store(bc, pipe.pop(0))

        start_gather(0, 0)

        def pair(p, _):
            for b in range(2):
                s = 2 * p + b
                wait_gather(b)

                @pl.when(s + 1 < S)
                def _():
                    start_gather(1 - b, s + 1)

                @pl.when(s >= 2)
                def _():
                    wait_out(b)

                transpose(b)
                start_out(b, s)
            return ()

        lax.fori_loop(0, S // 2, pair, ())
        wait_out(0)
        wait_out(1)

    return gather_kernel


def kernel(input_ids, table):
    batch, seq = input_ids.shape
    V, D = table.shape
    ids3 = input_ids.T.reshape(seq, batch // _BC, _BC).astype(jnp.int32)
    out2 = _gather_call(seq, V)(table, ids3)
    out5 = out2.reshape(seq, _TD, batch // _BC, 8, _BC)
    return out5.transpose(2, 4, 0, 1, 3).reshape(batch, seq, D)
